# baseline (speedup 1.0000x reference)
import jax
import jax.numpy as jnp
from jax.experimental import pallas as pl


def _maxes_body(xa_ref, xb_ref, out_ref):
    i = pl.program_id(0)
    n_steps = pl.num_programs(0)
    C2 = out_ref.shape[1]
    C = C2 // 2
    m = xa_ref.shape[-1]
    row = jax.lax.broadcasted_iota(jnp.int32, (m, m), 0)
    col = jax.lax.broadcasted_iota(jnp.int32, (m, m), 1)
    eq = (row == col)[None]
    neg = jnp.float32(-jnp.inf)
    for half, x in enumerate((xa_ref[...], xb_ref[...])):
        N = x.shape[0]
        per_row = C // N
        dmax = jnp.max(jnp.where(eq, x, neg), axis=(1, 2)).reshape(1, N)
        omax = jnp.max(jnp.where(eq, neg, x), axis=(1, 2)).reshape(1, N)
        for step in range(n_steps):
            gstep = half * n_steps + step
            b = gstep // per_row
            c0 = (gstep % per_row) * N

            @pl.when(i == step)
            def _(b=b, c0=c0, dmax=dmax, omax=omax):
                out_ref[b : b + 1, c0 : c0 + N] = dmax
                out_ref[b : b + 1, C + c0 : C + c0 + N] = omax


def kernel(x):
    B, C, m, _ = x.shape
    n_mat = B * C
    N = 8
    xr = x.reshape(n_mat, m, m)
    steps = n_mat // (2 * N)
    return pl.pallas_call(
        _maxes_body,
        grid=(steps,),
        in_specs=[
            pl.BlockSpec((N, m, m), lambda i: (i, 0, 0)),
            pl.BlockSpec((N, m, m), lambda i, s=steps: (i + s, 0, 0)),
        ],
        out_specs=pl.BlockSpec((B, 2 * C), lambda i: (0, 0)),
        out_shape=jax.ShapeDtypeStruct((B, 2 * C), x.dtype),
    )(xr, xr)


# speedup vs baseline: 1.0297x; 1.0297x over previous
import jax
import jax.numpy as jnp
from jax.experimental import pallas as pl


def _maxes_body(xa_ref, xb_ref, out_ref):
    i = pl.program_id(0)
    n_steps = pl.num_programs(0)
    C2 = out_ref.shape[1]
    C = C2 // 2
    m = xa_ref.shape[-1]
    row = jax.lax.broadcasted_iota(jnp.int32, (m, m), 0)
    col = jax.lax.broadcasted_iota(jnp.int32, (m, m), 1)
    eq = (row == col)[None]
    neg = jnp.float32(-jnp.inf)
    for half, x in enumerate((xa_ref[...], xb_ref[...])):
        N = x.shape[0]
        per_row = C // N
        dmax = jnp.max(jnp.where(eq, x, neg), axis=(1, 2)).reshape(1, N)
        omax = jnp.max(jnp.where(eq, neg, x), axis=(1, 2)).reshape(1, N)
        for step in range(n_steps):
            gstep = half * n_steps + step
            b = gstep // per_row
            c0 = (gstep % per_row) * N

            @pl.when(i == step)
            def _(b=b, c0=c0, dmax=dmax, omax=omax):
                out_ref[b : b + 1, c0 : c0 + N] = dmax
                out_ref[b : b + 1, C + c0 : C + c0 + N] = omax


def kernel(x):
    B, C, m, _ = x.shape
    n_mat = B * C
    N = 4
    xr = x.reshape(n_mat, m, m)
    steps = n_mat // (2 * N)
    return pl.pallas_call(
        _maxes_body,
        grid=(steps,),
        in_specs=[
            pl.BlockSpec((N, m, m), lambda i: (i, 0, 0)),
            pl.BlockSpec((N, m, m), lambda i, s=steps: (i + s, 0, 0)),
        ],
        out_specs=pl.BlockSpec((B, 2 * C), lambda i: (0, 0)),
        out_shape=jax.ShapeDtypeStruct((B, 2 * C), x.dtype),
    )(xr, xr)
